# baseline (speedup 1.0000x reference)
import functools

import jax
import jax.numpy as jnp
from jax import lax
from jax.experimental import pallas as pl
from jax.experimental.pallas import tpu as pltpu
from jax.experimental.pallas import tpu_sc as plsc

_VOCAB = 100000
_CTX = 2048
_DIM = 1024
_BATCH = 4

_NC = 2
_NS = 16
_NW = _NC * _NS
_PW = _CTX // _NW
_SUB = 8
_NSTEP = _PW // _SUB
_STEPS = _BATCH * _NSTEP
_NBUF = 7
_PRIME = 4
_LANES = 16


def _body(x_hbm, tok_hbm, pos_hbm, out_hbm, idx_v, pos_v, *ring):
    rows = ring[:_NBUF]
    gsem = ring[_NBUF:2 * _NBUF]
    osem = ring[2 * _NBUF:]

    wid = lax.axis_index("s") * _NC + lax.axis_index("c")
    p0 = wid * _PW

    for b in range(_BATCH):
        pltpu.sync_copy(x_hbm.at[b, pl.ds(p0, _PW)], idx_v.at[b])
    pltpu.sync_copy(pos_hbm.at[pl.ds(p0, _PW)], pos_v)

    gd = {}
    od = {}

    def gather(s):
        b, c = divmod(s, _NSTEP)
        gd[s] = pltpu.async_copy(
            tok_hbm.at[idx_v.at[b, pl.ds(c * _SUB, _SUB)]],
            rows[s % _NBUF], gsem[s % _NBUF])

    def outcopy(s):
        b, c = divmod(s, _NSTEP)
        od[s] = pltpu.async_copy(
            rows[s % _NBUF],
            out_hbm.at[b, pl.ds(p0 + c * _SUB, _SUB)], osem[s % _NBUF])

    def add_pos(s):
        c = s % _NSTEP
        buf = rows[s % _NBUF]
        nchunk = _DIM // _LANES

        def add_chunk(j, _):
            r = j // nchunk
            sl = pl.ds((j % nchunk) * _LANES, _LANES)
            plsc.addupdate(buf.at[r, sl], pos_v[c * _SUB + r, sl])
            return 0

        lax.fori_loop(0, _SUB * nchunk, add_chunk, 0, unroll=8)

    for s in range(_PRIME):
        gather(s)
    for s in range(_STEPS):
        k = s + _PRIME
        if k < _STEPS:
            if k >= _NBUF:
                od[k - _NBUF].wait()
            gather(k)
        gd[s].wait()
        add_pos(s)
        outcopy(s)
    for s in range(_STEPS - _NBUF, _STEPS):
        od[s].wait()


def kernel(x, token_table, pos_table):
    mesh = plsc.VectorSubcoreMesh(core_axis_name="c", subcore_axis_name="s")
    run = functools.partial(
        pl.kernel,
        mesh=mesh,
        out_type=jax.ShapeDtypeStruct((_BATCH, _CTX, _DIM), jnp.float32),
        scratch_types=(
            [pltpu.VMEM((_BATCH, _PW), jnp.int32),
             pltpu.VMEM((_PW, _DIM), jnp.float32)]
            + [pltpu.VMEM((_SUB, _DIM), jnp.float32)] * _NBUF
            + [pltpu.SemaphoreType.DMA] * (2 * _NBUF)
        ),
    )(_body)
    return run(x, token_table, pos_table)
